# baseline (speedup 1.0000x reference)
import jax
import jax.numpy as jnp
from jax import lax
from jax.experimental import pallas as pl
from jax.experimental.pallas import tpu as pltpu

_NN = 29
_NP = 32
_NJ = 16
_ND = 3
_INF = 6
_CTX = 2
_H = 64
_L = 4
_NORM = 100.0
_B = 8

_INTERPRET = False


def _silu(z):
    return z * (1.0 / (1.0 + jnp.exp(-z)))


def _body(*refs):
    x0_ref, hc_ref, nm_ref, em_ref, vld_ref = refs[:5]
    out_ref = refs[-1]
    prefs = refs[5:-1]

    B, NP, NJ, H = _B, _NP, _NJ, _H
    Bn = B * NP
    Eh = B * NP * NJ
    f32 = jnp.float32

    def bc(v, w):
        return jnp.broadcast_to(v, (Eh, w))

    x0 = x0_ref[...].reshape(Bn, _ND)
    nmf = nm_ref[...].reshape(Bn, 1)
    emf2 = em_ref[...].reshape(Eh, 2)
    vld2 = jnp.broadcast_to(vld_ref[...][None], (B, NP * NJ, 2)).reshape(Eh, 2)
    em128 = jnp.concatenate([bc(emf2[:, :1], H), bc(emf2[:, 1:], H)], axis=1)
    vld6 = jnp.concatenate([bc(vld2[:, :1], _ND), bc(vld2[:, 1:], _ND)], axis=1)

    hc = hc_ref[...].reshape(Bn, _INF + 1 + _CTX)
    ew = prefs[0][...]
    eb = prefs[1][...]
    h = (jnp.dot(hc, ew, preferred_element_type=f32) + eb) * nmf
    x = x0

    idx = 2
    for _l in range(_L):
        (e1a, e1b, e1dd, e1bias, e2wd, e2bd, c1wd, c1bd, c2wd, c2b,
         n1a, n1b, n1bias, n2w, n2b) = (p[...] for p in prefs[idx:idx + 15])
        idx += 15

        ai = (jnp.dot(h, e1a, preferred_element_type=f32) + e1bias).reshape(B, NP, H)
        ai128 = jnp.concatenate([ai, ai], axis=2)
        bj3 = jnp.dot(h, e1b, preferred_element_type=f32).reshape(B, NP, H)
        bj2 = jnp.concatenate([bj3[:, :NJ, :], bj3[:, NJ:, :]], axis=2)
        pre = (lax.broadcast_in_dim(ai128, (B, NP, NJ, 2 * H), (0, 1, 3)) +
               lax.broadcast_in_dim(bj2, (B, NP, NJ, 2 * H), (0, 2, 3))
               ).reshape(Eh, 2 * H)

        x3 = x.reshape(B, NP, _ND)
        xx6 = jnp.concatenate([x3, x3], axis=2)
        xj6 = jnp.concatenate([x3[:, :NJ, :], x3[:, NJ:, :]], axis=2)
        dif = (lax.broadcast_in_dim(xx6, (B, NP, NJ, 2 * _ND), (0, 1, 3)) -
               lax.broadcast_in_dim(xj6, (B, NP, NJ, 2 * _ND), (0, 2, 3))
               ).reshape(Eh, 2 * _ND)
        s = dif * dif
        d2e = jnp.sum(s[:, :_ND], axis=1, keepdims=True)
        d2o = jnp.sum(s[:, _ND:], axis=1, keepdims=True)
        pre = pre + jnp.concatenate([bc(d2e, H), bc(d2o, H)], axis=1) * e1dd

        m = _silu(jnp.dot(_silu(pre), e2wd, preferred_element_type=f32) + e2bd) * em128
        cc = _silu(jnp.dot(m, c1wd, preferred_element_type=f32) + c1bd)
        cw = cc * c2wd
        ce = jnp.sum(cw[:, :H], axis=1, keepdims=True) + c2b
        co = jnp.sum(cw[:, H:], axis=1, keepdims=True) + c2b
        cd = dif / jnp.concatenate([bc(jnp.sqrt(d2e + 1e-8), _ND),
                                    bc(jnp.sqrt(d2o + 1e-8), _ND)], axis=1)
        c6 = jnp.concatenate([bc(ce, _ND), bc(co, _ND)], axis=1) * vld6
        trans = cd * c6

        aggx6 = jnp.sum(trans.reshape(Bn, NJ, 2 * _ND), axis=1)
        aggx = (aggx6[:, :_ND] + aggx6[:, _ND:]) * (1.0 / _NORM)
        x = (x + aggx) * nmf
        aggm2 = jnp.sum(m.reshape(Bn, NJ, 2 * H), axis=1)
        aggm = (aggm2[:, :H] + aggm2[:, H:]) * (1.0 / _NORM)

        npre = (jnp.dot(h, n1a, preferred_element_type=f32) +
                jnp.dot(aggm, n1b, preferred_element_type=f32) + n1bias)
        h = (h + jnp.dot(_silu(npre), n2w, preferred_element_type=f32) + n2b) * nmf

    ow = prefs[idx][...]
    ob = prefs[idx + 1][...]
    hout = (jnp.dot(h, ow, preferred_element_type=f32) + ob) * nmf
    hf = hout[:, :_INF]

    vel = (x - x0) * nmf
    v3 = vel.reshape(B, NP, _ND)
    nm3 = nmf.reshape(B, NP, 1)
    cnt = jnp.sum(nm3, axis=1, keepdims=True)
    mean = jnp.sum(v3 * nm3, axis=1, keepdims=True) / cnt
    v3 = (v3 - mean) * nm3
    out_ref[...] = jnp.concatenate([v3, hf.reshape(B, NP, _INF)], axis=2)


def kernel(t, xh, node_mask, edge_mask, context, params):
    bs, n, dims = xh.shape
    f32 = jnp.float32
    nm = node_mask.astype(f32)
    xm = xh * nm
    x0 = xm[:, :, :_ND]
    tcol = jnp.full((bs, n, 1), t[0], f32)
    hcat = jnp.concatenate([xm[:, :, _ND:], tcol, context], axis=2)

    pad = _NP - n
    x0p = jnp.pad(x0, ((0, 0), (0, pad), (0, 0)))
    hcp = jnp.pad(hcat, ((0, 0), (0, pad), (0, 0)))
    nmp = jnp.pad(nm, ((0, 0), (0, pad), (0, 0)))
    em32 = jnp.pad(edge_mask.reshape(bs, n, n), ((0, 0), (0, pad), (0, pad)))
    emp = jnp.stack([em32[:, :, :_NJ], em32[:, :, _NJ:]], axis=-1)
    emp = emp.reshape(bs, _NP * _NJ, 2)
    vi = (jnp.arange(_NP) < n)
    vm = (vi[:, None] & vi[None, :]).astype(f32)
    vld = jnp.stack([vm[:, :_NJ], vm[:, _NJ:]], axis=-1).reshape(_NP * _NJ, 2)

    H = _H
    eye2 = jnp.eye(2, dtype=f32)

    def pair_w(w):
        return jnp.kron(eye2, w)

    def pair_b(b):
        return jnp.concatenate([b.reshape(1, -1), b.reshape(1, -1)], axis=1)

    plist = [params['emb'][0], params['emb'][1].reshape(1, H)]
    for l in range(_L):
        w1, b1 = params['e1_%d' % l]
        w2, b2 = params['e2_%d' % l]
        cw1, cb1 = params['c1_%d' % l]
        cw2, cb2 = params['c2_%d' % l]
        nw1, nb1 = params['n1_%d' % l]
        nw2, nb2 = params['n2_%d' % l]
        plist += [w1[:H], w1[H:2 * H], pair_b(w1[2 * H:]), b1.reshape(1, H),
                  pair_w(w2), pair_b(b2),
                  pair_w(cw1), pair_b(cb1),
                  pair_b(cw2.reshape(1, H)), cb2.reshape(1, 1),
                  nw1[:H], nw1[H:], nb1.reshape(1, H),
                  nw2, nb2.reshape(1, H)]
    plist += [params['out'][0], params['out'][1].reshape(1, dims)]

    data_specs = [
        pl.BlockSpec((_B, _NP, _ND), lambda i: (i, 0, 0)),
        pl.BlockSpec((_B, _NP, _INF + 1 + _CTX), lambda i: (i, 0, 0)),
        pl.BlockSpec((_B, _NP, 1), lambda i: (i, 0, 0)),
        pl.BlockSpec((_B, _NP * _NJ, 2), lambda i: (i, 0, 0)),
        pl.BlockSpec((_NP * _NJ, 2), lambda i: (0, 0)),
    ]
    param_specs = [
        pl.BlockSpec(p.shape, (lambda nd: lambda i: (0,) * nd)(p.ndim))
        for p in plist
    ]

    out = pl.pallas_call(
        _body,
        grid=(bs // _B,),
        in_specs=data_specs + param_specs,
        out_specs=pl.BlockSpec((_B, _NP, dims), lambda i: (i, 0, 0)),
        out_shape=jax.ShapeDtypeStruct((bs, _NP, dims), f32),
        compiler_params=pltpu.CompilerParams(
            dimension_semantics=("parallel",),
            vmem_limit_bytes=100 * 1024 * 1024,
        ),
        interpret=_INTERPRET,
    )(x0p, hcp, nmp, emp, vld, *plist)
    return out[:, :n, :]


# speedup vs baseline: 1.2492x; 1.2492x over previous
import jax
import jax.numpy as jnp
from jax import lax
from jax.experimental import pallas as pl
from jax.experimental.pallas import tpu as pltpu

_NN = 29
_NP = 32
_ND = 3
_INF = 6
_CTX = 2
_H = 64
_L = 4
_NORM = 100.0
_B = 8

_INTERPRET = False


def _silu(z):
    return z * (1.0 / (1.0 + jnp.exp(-z)))


def _body(*refs):
    x0_ref, hc_ref, nm_ref, em_ref, vld_ref = refs[:5]
    out_ref = refs[-1]
    prefs = refs[5:-1]

    B, NP, H = _B, _NP, _H
    Bn = B * NP
    E = B * NP * NP
    f32 = jnp.float32

    x0 = x0_ref[...].reshape(Bn, _ND)
    nmf = nm_ref[...].reshape(Bn, 1)
    emf = em_ref[...].reshape(E, 1)
    vld = jnp.broadcast_to(vld_ref[...][None], (B, NP * NP, 1)).reshape(E, 1)

    hc = hc_ref[...].reshape(Bn, _INF + 1 + _CTX)
    ew = prefs[0][...]
    eb = prefs[1][...]
    h = (jnp.dot(hc, ew, preferred_element_type=f32) + eb) * nmf
    x = x0

    idx = 2
    for _l in range(_L):
        (e1a, e1b, e1d, e1bias, e2w, e2b, c1w, c1b, c2w, c2b,
         n1a, n1b, n1bias, n2w, n2b) = (p[...] for p in prefs[idx:idx + 15])
        idx += 15

        ai = (jnp.dot(h, e1a, preferred_element_type=f32) + e1bias).reshape(B, NP, H)
        bj = jnp.dot(h, e1b, preferred_element_type=f32).reshape(B, NP, H)
        pre = (lax.broadcast_in_dim(ai, (B, NP, NP, H), (0, 1, 3)) +
               lax.broadcast_in_dim(bj, (B, NP, NP, H), (0, 2, 3))).reshape(E, H)

        x3 = x.reshape(B, NP, _ND)
        dif = (lax.broadcast_in_dim(x3, (B, NP, NP, _ND), (0, 1, 3)) -
               lax.broadcast_in_dim(x3, (B, NP, NP, _ND), (0, 2, 3))).reshape(E, _ND)
        d2 = jnp.sum(dif * dif, axis=1, keepdims=True)
        pre = pre + d2 * e1d

        m = _silu(jnp.dot(_silu(pre), e2w, preferred_element_type=f32) + e2b) * emf
        cc = _silu(jnp.dot(m, c1w, preferred_element_type=f32) + c1b)
        c = jnp.sum(cc * c2w, axis=1, keepdims=True) + c2b
        w = (c * vld) * lax.rsqrt(d2 + 1e-8)
        trans = dif * w

        aggx = jnp.sum(trans.reshape(Bn, NP, _ND), axis=1) * (1.0 / _NORM)
        x = (x + aggx) * nmf
        aggm = jnp.sum(m.reshape(Bn, NP, H), axis=1) * (1.0 / _NORM)

        npre = (jnp.dot(h, n1a, preferred_element_type=f32) +
                jnp.dot(aggm, n1b, preferred_element_type=f32) + n1bias)
        h = (h + jnp.dot(_silu(npre), n2w, preferred_element_type=f32) + n2b) * nmf

    ow = prefs[idx][...]
    ob = prefs[idx + 1][...]
    hout = (jnp.dot(h, ow, preferred_element_type=f32) + ob) * nmf
    hf = hout[:, :_INF]

    vel = (x - x0) * nmf
    v3 = vel.reshape(B, NP, _ND)
    nm3 = nmf.reshape(B, NP, 1)
    cnt = jnp.sum(nm3, axis=1, keepdims=True)
    mean = jnp.sum(v3 * nm3, axis=1, keepdims=True) / cnt
    v3 = (v3 - mean) * nm3
    out_ref[...] = jnp.concatenate([v3, hf.reshape(B, NP, _INF)], axis=2)


def kernel(t, xh, node_mask, edge_mask, context, params):
    bs, n, dims = xh.shape
    f32 = jnp.float32
    nm = node_mask.astype(f32)
    xm = xh * nm
    x0 = xm[:, :, :_ND]
    tcol = jnp.full((bs, n, 1), t[0], f32)
    hcat = jnp.concatenate([xm[:, :, _ND:], tcol, context], axis=2)

    pad = _NP - n
    x0p = jnp.pad(x0, ((0, 0), (0, pad), (0, 0)))
    hcp = jnp.pad(hcat, ((0, 0), (0, pad), (0, 0)))
    nmp = jnp.pad(nm, ((0, 0), (0, pad), (0, 0)))
    emp = jnp.pad(edge_mask.reshape(bs, n, n), ((0, 0), (0, pad), (0, pad)))
    emp = emp.reshape(bs, _NP * _NP, 1)
    vi = (jnp.arange(_NP) < n)
    vld = (vi[:, None] & vi[None, :]).astype(f32).reshape(_NP * _NP, 1)

    H = _H
    plist = [params['emb'][0], params['emb'][1].reshape(1, H)]
    for l in range(_L):
        w1, b1 = params['e1_%d' % l]
        w2, b2 = params['e2_%d' % l]
        cw1, cb1 = params['c1_%d' % l]
        cw2, cb2 = params['c2_%d' % l]
        nw1, nb1 = params['n1_%d' % l]
        nw2, nb2 = params['n2_%d' % l]
        plist += [w1[:H], w1[H:2 * H], w1[2 * H:], b1.reshape(1, H),
                  w2, b2.reshape(1, H),
                  cw1, cb1.reshape(1, H),
                  cw2.reshape(1, H), cb2.reshape(1, 1),
                  nw1[:H], nw1[H:], nb1.reshape(1, H),
                  nw2, nb2.reshape(1, H)]
    plist += [params['out'][0], params['out'][1].reshape(1, dims)]

    data_specs = [
        pl.BlockSpec((_B, _NP, _ND), lambda i: (i, 0, 0)),
        pl.BlockSpec((_B, _NP, _INF + 1 + _CTX), lambda i: (i, 0, 0)),
        pl.BlockSpec((_B, _NP, 1), lambda i: (i, 0, 0)),
        pl.BlockSpec((_B, _NP * _NP, 1), lambda i: (i, 0, 0)),
        pl.BlockSpec((_NP * _NP, 1), lambda i: (0, 0)),
    ]
    param_specs = [
        pl.BlockSpec(p.shape, (lambda nd: lambda i: (0,) * nd)(p.ndim))
        for p in plist
    ]

    out = pl.pallas_call(
        _body,
        grid=(bs // _B,),
        in_specs=data_specs + param_specs,
        out_specs=pl.BlockSpec((_B, _NP, dims), lambda i: (i, 0, 0)),
        out_shape=jax.ShapeDtypeStruct((bs, _NP, dims), f32),
        compiler_params=pltpu.CompilerParams(
            dimension_semantics=("parallel",),
            vmem_limit_bytes=100 * 1024 * 1024,
        ),
        interpret=_INTERPRET,
    )(x0p, hcp, nmp, emp, vld, *plist)
    return out[:, :n, :]


# speedup vs baseline: 1.3024x; 1.0426x over previous
import jax
import jax.numpy as jnp
from jax import lax
from jax.experimental import pallas as pl
from jax.experimental.pallas import tpu as pltpu

_NN = 29
_NP = 32
_ND = 3
_INF = 6
_CTX = 2
_H = 64
_L = 4
_NORM = 100.0
_B = 8

_INTERPRET = False


_LOG2E = 1.4426950408889634


def _silu(z):
    return z * (1.0 / (1.0 + jnp.exp2(z * -_LOG2E)))


def _body(*refs):
    x0_ref, hc_ref, nm_ref, em_ref, vld_ref = refs[:5]
    out_ref = refs[-1]
    prefs = refs[5:-1]

    B, NP, H = _B, _NP, _H
    Bn = B * NP
    E = B * NP * NP
    f32 = jnp.float32

    x0 = x0_ref[...].reshape(Bn, _ND)
    nmf = nm_ref[...].reshape(Bn, 1)
    emf = em_ref[...].reshape(E, 1)
    vld = jnp.broadcast_to(vld_ref[...][None], (B, NP * NP, 1)).reshape(E, 1)

    hc = hc_ref[...].reshape(Bn, _INF + 1 + _CTX)
    ew = prefs[0][...]
    eb = prefs[1][...]
    h = (jnp.dot(hc, ew, preferred_element_type=f32) + eb) * nmf
    x = x0

    idx = 2
    for _l in range(_L):
        (e1a, e1b, e1d, e1bias, e2w, e2b, c1w, c1b, c2w, c2b,
         n1a, n1b, n1bias, n2w, n2b) = (p[...] for p in prefs[idx:idx + 15])
        idx += 15

        ai = (jnp.dot(h, e1a, preferred_element_type=f32) + e1bias).reshape(B, NP, H)
        bj = jnp.dot(h, e1b, preferred_element_type=f32).reshape(B, NP, H)
        pre = (lax.broadcast_in_dim(ai, (B, NP, NP, H), (0, 1, 3)) +
               lax.broadcast_in_dim(bj, (B, NP, NP, H), (0, 2, 3))).reshape(E, H)

        x3 = x.reshape(B, NP, _ND)
        dif = (lax.broadcast_in_dim(x3, (B, NP, NP, _ND), (0, 1, 3)) -
               lax.broadcast_in_dim(x3, (B, NP, NP, _ND), (0, 2, 3))).reshape(E, _ND)
        d2 = jnp.sum(dif * dif, axis=1, keepdims=True)
        pre = pre + d2 * e1d

        m = _silu(jnp.dot(_silu(pre), e2w, preferred_element_type=f32) + e2b) * emf
        cc = _silu(jnp.dot(m, c1w, preferred_element_type=f32) + c1b)
        c = jnp.sum(cc * c2w, axis=1, keepdims=True) + c2b
        w = (c * vld) * lax.rsqrt(d2 + 1e-8)
        trans = dif * w

        aggx = jnp.sum(trans.reshape(Bn, NP, _ND), axis=1) * (1.0 / _NORM)
        x = (x + aggx) * nmf
        aggm = jnp.sum(m.reshape(Bn, NP, H), axis=1) * (1.0 / _NORM)

        npre = (jnp.dot(h, n1a, preferred_element_type=f32) +
                jnp.dot(aggm, n1b, preferred_element_type=f32) + n1bias)
        h = (h + jnp.dot(_silu(npre), n2w, preferred_element_type=f32) + n2b) * nmf

    ow = prefs[idx][...]
    ob = prefs[idx + 1][...]
    hout = (jnp.dot(h, ow, preferred_element_type=f32) + ob) * nmf
    hf = hout[:, :_INF]

    vel = (x - x0) * nmf
    v3 = vel.reshape(B, NP, _ND)
    nm3 = nmf.reshape(B, NP, 1)
    cnt = jnp.sum(nm3, axis=1, keepdims=True)
    mean = jnp.sum(v3 * nm3, axis=1, keepdims=True) / cnt
    v3 = (v3 - mean) * nm3
    out_ref[...] = jnp.concatenate([v3, hf.reshape(B, NP, _INF)], axis=2)


def kernel(t, xh, node_mask, edge_mask, context, params):
    bs, n, dims = xh.shape
    f32 = jnp.float32
    nm = node_mask.astype(f32)
    xm = xh * nm
    x0 = xm[:, :, :_ND]
    tcol = jnp.full((bs, n, 1), t[0], f32)
    hcat = jnp.concatenate([xm[:, :, _ND:], tcol, context], axis=2)

    pad = _NP - n
    x0p = jnp.pad(x0, ((0, 0), (0, pad), (0, 0)))
    hcp = jnp.pad(hcat, ((0, 0), (0, pad), (0, 0)))
    nmp = jnp.pad(nm, ((0, 0), (0, pad), (0, 0)))
    emp = jnp.pad(edge_mask.reshape(bs, n, n), ((0, 0), (0, pad), (0, pad)))
    emp = emp.reshape(bs, _NP * _NP, 1)
    vi = (jnp.arange(_NP) < n)
    vld = (vi[:, None] & vi[None, :]).astype(f32).reshape(_NP * _NP, 1)

    H = _H
    plist = [params['emb'][0], params['emb'][1].reshape(1, H)]
    for l in range(_L):
        w1, b1 = params['e1_%d' % l]
        w2, b2 = params['e2_%d' % l]
        cw1, cb1 = params['c1_%d' % l]
        cw2, cb2 = params['c2_%d' % l]
        nw1, nb1 = params['n1_%d' % l]
        nw2, nb2 = params['n2_%d' % l]
        plist += [w1[:H], w1[H:2 * H], w1[2 * H:], b1.reshape(1, H),
                  w2, b2.reshape(1, H),
                  cw1, cb1.reshape(1, H),
                  cw2.reshape(1, H), cb2.reshape(1, 1),
                  nw1[:H], nw1[H:], nb1.reshape(1, H),
                  nw2, nb2.reshape(1, H)]
    plist += [params['out'][0], params['out'][1].reshape(1, dims)]

    data_specs = [
        pl.BlockSpec((_B, _NP, _ND), lambda i: (i, 0, 0)),
        pl.BlockSpec((_B, _NP, _INF + 1 + _CTX), lambda i: (i, 0, 0)),
        pl.BlockSpec((_B, _NP, 1), lambda i: (i, 0, 0)),
        pl.BlockSpec((_B, _NP * _NP, 1), lambda i: (i, 0, 0)),
        pl.BlockSpec((_NP * _NP, 1), lambda i: (0, 0)),
    ]
    param_specs = [
        pl.BlockSpec(p.shape, (lambda nd: lambda i: (0,) * nd)(p.ndim))
        for p in plist
    ]

    out = pl.pallas_call(
        _body,
        grid=(bs // _B,),
        in_specs=data_specs + param_specs,
        out_specs=pl.BlockSpec((_B, _NP, dims), lambda i: (i, 0, 0)),
        out_shape=jax.ShapeDtypeStruct((bs, _NP, dims), f32),
        compiler_params=pltpu.CompilerParams(
            dimension_semantics=("parallel",),
            vmem_limit_bytes=100 * 1024 * 1024,
        ),
        interpret=_INTERPRET,
    )(x0p, hcp, nmp, emp, vld, *plist)
    return out[:, :n, :]


# speedup vs baseline: 1.3302x; 1.0213x over previous
import jax
import jax.numpy as jnp
from jax import lax
from jax.experimental import pallas as pl
from jax.experimental.pallas import tpu as pltpu

_NN = 29
_NP = 32
_ND = 3
_INF = 6
_CTX = 2
_H = 64
_L = 4
_NORM = 100.0
_B = 8

_INTERPRET = False


def _silu(z):
    return z * (0.5 * jnp.tanh(0.5 * z) + 0.5)


def _body(*refs):
    x0_ref, hc_ref, nm_ref, em_ref, vld_ref = refs[:5]
    out_ref = refs[-1]
    prefs = refs[5:-1]

    B, NP, H = _B, _NP, _H
    Bn = B * NP
    E = B * NP * NP
    f32 = jnp.float32

    x0 = x0_ref[...].reshape(Bn, _ND)
    nmf = nm_ref[...].reshape(Bn, 1)
    emf = em_ref[...].reshape(E, 1)
    vld = jnp.broadcast_to(vld_ref[...][None], (B, NP * NP, 1)).reshape(E, 1)

    hc = hc_ref[...].reshape(Bn, _INF + 1 + _CTX)
    ew = prefs[0][...]
    eb = prefs[1][...]
    h = (jnp.dot(hc, ew, preferred_element_type=f32) + eb) * nmf
    x = x0

    idx = 2
    for _l in range(_L):
        (e1a, e1b, e1d, e1bias, e2w, e2b, c1w, c1b, c2w, c2b,
         n1a, n1b, n1bias, n2w, n2b) = (p[...] for p in prefs[idx:idx + 15])
        idx += 15

        ai = (jnp.dot(h, e1a, preferred_element_type=f32) + e1bias).reshape(B, NP, H)
        bj = jnp.dot(h, e1b, preferred_element_type=f32).reshape(B, NP, H)
        pre = (lax.broadcast_in_dim(ai, (B, NP, NP, H), (0, 1, 3)) +
               lax.broadcast_in_dim(bj, (B, NP, NP, H), (0, 2, 3))).reshape(E, H)

        x3 = x.reshape(B, NP, _ND)
        dif = (lax.broadcast_in_dim(x3, (B, NP, NP, _ND), (0, 1, 3)) -
               lax.broadcast_in_dim(x3, (B, NP, NP, _ND), (0, 2, 3))).reshape(E, _ND)
        d2 = jnp.sum(dif * dif, axis=1, keepdims=True)
        pre = pre + d2 * e1d

        m = _silu(jnp.dot(_silu(pre), e2w, preferred_element_type=f32) + e2b) * emf
        cc = _silu(jnp.dot(m, c1w, preferred_element_type=f32) + c1b)
        c = jnp.sum(cc * c2w, axis=1, keepdims=True) + c2b
        w = (c * vld) * lax.rsqrt(d2 + 1e-8)
        trans = dif * w

        aggx = jnp.sum(trans.reshape(Bn, NP, _ND), axis=1) * (1.0 / _NORM)
        x = (x + aggx) * nmf
        aggm = jnp.sum(m.reshape(Bn, NP, H), axis=1) * (1.0 / _NORM)

        npre = (jnp.dot(h, n1a, preferred_element_type=f32) +
                jnp.dot(aggm, n1b, preferred_element_type=f32) + n1bias)
        h = (h + jnp.dot(_silu(npre), n2w, preferred_element_type=f32) + n2b) * nmf

    ow = prefs[idx][...]
    ob = prefs[idx + 1][...]
    hout = (jnp.dot(h, ow, preferred_element_type=f32) + ob) * nmf
    hf = hout[:, :_INF]

    vel = (x - x0) * nmf
    v3 = vel.reshape(B, NP, _ND)
    nm3 = nmf.reshape(B, NP, 1)
    cnt = jnp.sum(nm3, axis=1, keepdims=True)
    mean = jnp.sum(v3 * nm3, axis=1, keepdims=True) / cnt
    v3 = (v3 - mean) * nm3
    out_ref[...] = jnp.concatenate([v3, hf.reshape(B, NP, _INF)], axis=2)


def kernel(t, xh, node_mask, edge_mask, context, params):
    bs, n, dims = xh.shape
    f32 = jnp.float32
    nm = node_mask.astype(f32)
    xm = xh * nm
    x0 = xm[:, :, :_ND]
    tcol = jnp.full((bs, n, 1), t[0], f32)
    hcat = jnp.concatenate([xm[:, :, _ND:], tcol, context], axis=2)

    pad = _NP - n
    x0p = jnp.pad(x0, ((0, 0), (0, pad), (0, 0)))
    hcp = jnp.pad(hcat, ((0, 0), (0, pad), (0, 0)))
    nmp = jnp.pad(nm, ((0, 0), (0, pad), (0, 0)))
    emp = jnp.pad(edge_mask.reshape(bs, n, n), ((0, 0), (0, pad), (0, pad)))
    emp = emp.reshape(bs, _NP * _NP, 1)
    vi = (jnp.arange(_NP) < n)
    vld = (vi[:, None] & vi[None, :]).astype(f32).reshape(_NP * _NP, 1)

    H = _H
    plist = [params['emb'][0], params['emb'][1].reshape(1, H)]
    for l in range(_L):
        w1, b1 = params['e1_%d' % l]
        w2, b2 = params['e2_%d' % l]
        cw1, cb1 = params['c1_%d' % l]
        cw2, cb2 = params['c2_%d' % l]
        nw1, nb1 = params['n1_%d' % l]
        nw2, nb2 = params['n2_%d' % l]
        plist += [w1[:H], w1[H:2 * H], w1[2 * H:], b1.reshape(1, H),
                  w2, b2.reshape(1, H),
                  cw1, cb1.reshape(1, H),
                  cw2.reshape(1, H), cb2.reshape(1, 1),
                  nw1[:H], nw1[H:], nb1.reshape(1, H),
                  nw2, nb2.reshape(1, H)]
    plist += [params['out'][0], params['out'][1].reshape(1, dims)]

    data_specs = [
        pl.BlockSpec((_B, _NP, _ND), lambda i: (i, 0, 0)),
        pl.BlockSpec((_B, _NP, _INF + 1 + _CTX), lambda i: (i, 0, 0)),
        pl.BlockSpec((_B, _NP, 1), lambda i: (i, 0, 0)),
        pl.BlockSpec((_B, _NP * _NP, 1), lambda i: (i, 0, 0)),
        pl.BlockSpec((_NP * _NP, 1), lambda i: (0, 0)),
    ]
    param_specs = [
        pl.BlockSpec(p.shape, (lambda nd: lambda i: (0,) * nd)(p.ndim))
        for p in plist
    ]

    out = pl.pallas_call(
        _body,
        grid=(bs // _B,),
        in_specs=data_specs + param_specs,
        out_specs=pl.BlockSpec((_B, _NP, dims), lambda i: (i, 0, 0)),
        out_shape=jax.ShapeDtypeStruct((bs, _NP, dims), f32),
        compiler_params=pltpu.CompilerParams(
            dimension_semantics=("parallel",),
            vmem_limit_bytes=100 * 1024 * 1024,
        ),
        interpret=_INTERPRET,
    )(x0p, hcp, nmp, emp, vld, *plist)
    return out[:, :n, :]


# speedup vs baseline: 1.3458x; 1.0117x over previous
import jax
import jax.numpy as jnp
from jax import lax
from jax.experimental import pallas as pl
from jax.experimental.pallas import tpu as pltpu

_NN = 29
_NP = 32
_ND = 3
_INF = 6
_CTX = 2
_H = 64
_L = 4
_NORM = 100.0
_B = 4

_INTERPRET = False


def _silu(z):
    return z * (0.5 * jnp.tanh(0.5 * z) + 0.5)


def _body(*refs):
    x0_ref, hc_ref, nm_ref, em_ref, vld_ref = refs[:5]
    out_ref = refs[-1]
    prefs = refs[5:-1]

    B, NP, H = _B, _NP, _H
    Bn = B * NP
    E = B * NP * NP
    f32 = jnp.float32

    x0 = x0_ref[...].reshape(Bn, _ND)
    nmf = nm_ref[...].reshape(Bn, 1)
    emf = em_ref[...].reshape(E, 1)
    vld = jnp.broadcast_to(vld_ref[...][None], (B, NP * NP, 1)).reshape(E, 1)

    hc = hc_ref[...].reshape(Bn, _INF + 1 + _CTX)
    ew = prefs[0][...]
    eb = prefs[1][...]
    h = (jnp.dot(hc, ew, preferred_element_type=f32) + eb) * nmf
    x = x0

    idx = 2
    for _l in range(_L):
        (e1a, e1b, e1d, e1bias, e2w, e2b, c1w, c1b, c2w, c2b,
         n1a, n1b, n1bias, n2w, n2b) = (p[...] for p in prefs[idx:idx + 15])
        idx += 15

        ai = (jnp.dot(h, e1a, preferred_element_type=f32) + e1bias).reshape(B, NP, H)
        bj = jnp.dot(h, e1b, preferred_element_type=f32).reshape(B, NP, H)
        pre = (lax.broadcast_in_dim(ai, (B, NP, NP, H), (0, 1, 3)) +
               lax.broadcast_in_dim(bj, (B, NP, NP, H), (0, 2, 3))).reshape(E, H)

        x3 = x.reshape(B, NP, _ND)
        dif = (lax.broadcast_in_dim(x3, (B, NP, NP, _ND), (0, 1, 3)) -
               lax.broadcast_in_dim(x3, (B, NP, NP, _ND), (0, 2, 3))).reshape(E, _ND)
        d2 = jnp.sum(dif * dif, axis=1, keepdims=True)
        pre = pre + d2 * e1d

        m = _silu(jnp.dot(_silu(pre), e2w, preferred_element_type=f32) + e2b) * emf
        cc = _silu(jnp.dot(m, c1w, preferred_element_type=f32) + c1b)
        c = jnp.sum(cc * c2w, axis=1, keepdims=True) + c2b
        w = (c * vld) * lax.rsqrt(d2 + 1e-8)
        trans = dif * w

        aggx = jnp.sum(trans.reshape(Bn, NP, _ND), axis=1) * (1.0 / _NORM)
        x = (x + aggx) * nmf
        aggm = jnp.sum(m.reshape(Bn, NP, H), axis=1) * (1.0 / _NORM)

        npre = (jnp.dot(h, n1a, preferred_element_type=f32) +
                jnp.dot(aggm, n1b, preferred_element_type=f32) + n1bias)
        h = (h + jnp.dot(_silu(npre), n2w, preferred_element_type=f32) + n2b) * nmf

    ow = prefs[idx][...]
    ob = prefs[idx + 1][...]
    hout = (jnp.dot(h, ow, preferred_element_type=f32) + ob) * nmf
    hf = hout[:, :_INF]

    vel = (x - x0) * nmf
    v3 = vel.reshape(B, NP, _ND)
    nm3 = nmf.reshape(B, NP, 1)
    cnt = jnp.sum(nm3, axis=1, keepdims=True)
    mean = jnp.sum(v3 * nm3, axis=1, keepdims=True) / cnt
    v3 = (v3 - mean) * nm3
    out_ref[...] = jnp.concatenate([v3, hf.reshape(B, NP, _INF)], axis=2)


def kernel(t, xh, node_mask, edge_mask, context, params):
    bs, n, dims = xh.shape
    f32 = jnp.float32
    nm = node_mask.astype(f32)
    xm = xh * nm
    x0 = xm[:, :, :_ND]
    tcol = jnp.full((bs, n, 1), t[0], f32)
    hcat = jnp.concatenate([xm[:, :, _ND:], tcol, context], axis=2)

    pad = _NP - n
    x0p = jnp.pad(x0, ((0, 0), (0, pad), (0, 0)))
    hcp = jnp.pad(hcat, ((0, 0), (0, pad), (0, 0)))
    nmp = jnp.pad(nm, ((0, 0), (0, pad), (0, 0)))
    emp = jnp.pad(edge_mask.reshape(bs, n, n), ((0, 0), (0, pad), (0, pad)))
    emp = emp.reshape(bs, _NP * _NP, 1)
    vi = (jnp.arange(_NP) < n)
    vld = (vi[:, None] & vi[None, :]).astype(f32).reshape(_NP * _NP, 1)

    H = _H
    plist = [params['emb'][0], params['emb'][1].reshape(1, H)]
    for l in range(_L):
        w1, b1 = params['e1_%d' % l]
        w2, b2 = params['e2_%d' % l]
        cw1, cb1 = params['c1_%d' % l]
        cw2, cb2 = params['c2_%d' % l]
        nw1, nb1 = params['n1_%d' % l]
        nw2, nb2 = params['n2_%d' % l]
        plist += [w1[:H], w1[H:2 * H], w1[2 * H:], b1.reshape(1, H),
                  w2, b2.reshape(1, H),
                  cw1, cb1.reshape(1, H),
                  cw2.reshape(1, H), cb2.reshape(1, 1),
                  nw1[:H], nw1[H:], nb1.reshape(1, H),
                  nw2, nb2.reshape(1, H)]
    plist += [params['out'][0], params['out'][1].reshape(1, dims)]

    data_specs = [
        pl.BlockSpec((_B, _NP, _ND), lambda i: (i, 0, 0)),
        pl.BlockSpec((_B, _NP, _INF + 1 + _CTX), lambda i: (i, 0, 0)),
        pl.BlockSpec((_B, _NP, 1), lambda i: (i, 0, 0)),
        pl.BlockSpec((_B, _NP * _NP, 1), lambda i: (i, 0, 0)),
        pl.BlockSpec((_NP * _NP, 1), lambda i: (0, 0)),
    ]
    param_specs = [
        pl.BlockSpec(p.shape, (lambda nd: lambda i: (0,) * nd)(p.ndim))
        for p in plist
    ]

    out = pl.pallas_call(
        _body,
        grid=(bs // _B,),
        in_specs=data_specs + param_specs,
        out_specs=pl.BlockSpec((_B, _NP, dims), lambda i: (i, 0, 0)),
        out_shape=jax.ShapeDtypeStruct((bs, _NP, dims), f32),
        compiler_params=pltpu.CompilerParams(
            dimension_semantics=("parallel",),
            vmem_limit_bytes=100 * 1024 * 1024,
        ),
        interpret=_INTERPRET,
    )(x0p, hcp, nmp, emp, vld, *plist)
    return out[:, :n, :]
